# baseline (speedup 1.0000x reference)
import functools

import jax
import jax.numpy as jnp
from jax import lax
from jax.experimental import pallas as pl
from jax.experimental.pallas import tpu as pltpu


def _round_up(x, m):
    return ((x + m - 1) // m) * m


def _ffm_kernel(x_ref, wc_ref, bns_ref, bias_ref, ones_ref, w1_ref, w2_ref,
                out_ref, xpad_ref, *, H, W, B, G):
    HW = H * W
    Cin = x_ref.shape[1]
    Cout = out_ref.shape[1]

    col = lax.broadcasted_iota(jnp.int32, (1, HW), 1) % W
    left_ok = col != 0
    right_ok = col != W - 1

    wk = (wc_ref[...].reshape(9 * Cin, Cout)
          * (0.5 * bns_ref[...])).astype(jnp.bfloat16)
    bcb = bias_ref[:, 0:1]
    b1c = bias_ref[:, 1:2]
    b2c = bias_ref[:, 2:3]

    base = W + 1
    Lpad = xpad_ref.shape[3]
    for slot in range(2):
        for j in range(G):
            xpad_ref[slot, j, :, 0:base] = jnp.zeros((Cin, base),
                                                     jnp.bfloat16)
            xpad_ref[slot, j, :, base + HW:Lpad] = jnp.zeros(
                (Cin, Lpad - base - HW), jnp.bfloat16)

    for p in range(B // G):
        slot = p % 2
        for j in range(G):
            xpad_ref[slot, j, :, base:base + HW] = x_ref[G * p + j]

        slices = []
        for kh in range(3):
            for kw in range(3):
                o = kh * W + kw
                pieces = []
                for j in range(G):
                    s = xpad_ref[slot, j, :, o:o + HW]
                    if kw == 0:
                        s = jnp.where(left_ok, s, 0)
                    elif kw == 2:
                        s = jnp.where(right_ok, s, 0)
                    pieces.append(s)
                slices.append(jnp.concatenate(pieces, axis=1))
        rhs = jnp.concatenate(slices, axis=0)

        acc = lax.dot_general(
            wk, rhs,
            dimension_numbers=(((0,), (0,)), ((), ())),
            preferred_element_type=jnp.float32)
        feather = jnp.maximum((acc + bcb).astype(jnp.bfloat16), 0)

        pooled2 = jnp.dot(feather, ones_ref[...],
                          preferred_element_type=jnp.float32)
        h1 = jnp.maximum(
            lax.dot_general(w1_ref[...], pooled2,
                            dimension_numbers=(((0,), (0,)), ((), ())),
                            preferred_element_type=jnp.float32) + b1c, 0.0)
        z = lax.dot_general(w2_ref[...], h1,
                            dimension_numbers=(((0,), (0,)), ((), ())),
                            preferred_element_type=jnp.float32)
        for j in range(G):
            gate = 1.0 + jax.nn.sigmoid(z[:, 128 * j:128 * j + 1] + b2c)
            out_ref[G * p + j] = (feather[:, HW * j:HW * (j + 1)]
                                  * gate.astype(jnp.bfloat16))


@jax.jit
def _ffm(sp, cx, wc, bc, bns, bnb, w1, b1, w2, b2):
    N, Cin, H, W = sp.shape
    Cout = w1.shape[0]
    HW = H * W
    Lpad = _round_up(HW + 2 * W + 2, 128)
    B = next(b for b in (16, 8, 4, 2) if N % b == 0)
    G = 4 if B % 4 == 0 else 2

    bcb = bc.reshape(Cout) * bns.reshape(Cout) + bnb.reshape(Cout)
    bias_pack = jnp.stack([bcb, b1.reshape(Cout), b2.reshape(Cout)], axis=1)
    eyeg = jnp.eye(G, dtype=jnp.bfloat16)
    ones = jnp.kron(eyeg, jnp.full((HW, 128), 1.0 / HW, jnp.bfloat16))
    wc_r = wc.reshape(9 * Cin, Cout)

    x = (sp + cx).reshape(N, Cin, HW).astype(jnp.bfloat16)

    kernel_fn = functools.partial(_ffm_kernel, H=H, W=W, B=B, G=G)
    out = pl.pallas_call(
        kernel_fn,
        out_shape=jax.ShapeDtypeStruct((N, Cout, HW), jnp.bfloat16),
        grid=(N // B,),
        in_specs=[
            pl.BlockSpec((B, Cin, HW), lambda i: (i, 0, 0)),
            pl.BlockSpec((9 * Cin, Cout), lambda i: (0, 0)),
            pl.BlockSpec((1, Cout), lambda i: (0, 0)),
            pl.BlockSpec((Cout, 3), lambda i: (0, 0)),
            pl.BlockSpec((G * HW, 128 * G), lambda i: (0, 0)),
            pl.BlockSpec((Cout, Cout), lambda i: (0, 0)),
            pl.BlockSpec((Cout, Cout), lambda i: (0, 0)),
        ],
        out_specs=pl.BlockSpec((B, Cout, HW), lambda i: (i, 0, 0)),
        scratch_shapes=[pltpu.VMEM((2, G, Cin, Lpad), jnp.bfloat16)],
        compiler_params=pltpu.CompilerParams(
            dimension_semantics=("parallel",)),
        cost_estimate=pl.CostEstimate(
            flops=2 * N * 9 * Cout * Cin * HW + 2 * N * Cout * HW * 128
                  + 2 * N * 2 * Cout * Cout * 128,
            transcendentals=N * Cout,
            bytes_accessed=2 * (N * Cin * HW + N * Cout * HW)
                           + 4 * (2 * Cout * Cout + N * Cout)
                           + 4 * Cout * 9 * Cin),
    )(x, wc_r, bns, bias_pack, ones, w1, w2)

    return out.reshape(N, Cout, H, W).astype(jnp.float32)


def kernel(sp, cx, wc, bc, bns, bnb, w1, b1, w2, b2):
    return _ffm(sp, cx, wc, bc, bns, bnb, w1, b1, w2, b2)


# speedup vs baseline: 1.0621x; 1.0621x over previous
import functools

import jax
import jax.numpy as jnp
from jax import lax
from jax.experimental import pallas as pl
from jax.experimental.pallas import tpu as pltpu


def _round_up(x, m):
    return ((x + m - 1) // m) * m


def _ffm_kernel(x_ref, wc_ref, bns_ref, bias_ref, ones_ref, w1_ref, w2_ref,
                out_ref, xpad_ref, *, H, W, B, G):
    HW = H * W
    Cin = x_ref.shape[1]
    Cout = out_ref.shape[1]

    col = lax.broadcasted_iota(jnp.int32, (1, HW), 1) % W
    left_ok = col != 0
    right_ok = col != W - 1

    wk = (wc_ref[...].reshape(9 * Cin, Cout)
          * (0.5 * bns_ref[...])).astype(jnp.bfloat16)
    bcb = bias_ref[:, 0:1]
    b1c = bias_ref[:, 1:2]
    b2c = bias_ref[:, 2:3]

    base = W + 1
    Lpad = xpad_ref.shape[3]
    for slot in range(2):
        for j in range(G):
            xpad_ref[slot, j, :, 0:base] = jnp.zeros((Cin, base),
                                                     jnp.bfloat16)
            xpad_ref[slot, j, :, base + HW:Lpad] = jnp.zeros(
                (Cin, Lpad - base - HW), jnp.bfloat16)

    for p in range(B // G):
        slot = p % 2
        for j in range(G):
            xpad_ref[slot, j, :, base:base + HW] = x_ref[G * p + j]

        slices = []
        for kh in range(3):
            for kw in range(3):
                o = kh * W + kw
                pieces = []
                for j in range(G):
                    s = xpad_ref[slot, j, :, o:o + HW]
                    if kw == 0:
                        s = jnp.where(left_ok, s, 0)
                    elif kw == 2:
                        s = jnp.where(right_ok, s, 0)
                    pieces.append(s)
                slices.append(jnp.concatenate(pieces, axis=1))
        rhs = jnp.concatenate(slices, axis=0)

        acc = lax.dot_general(
            wk, rhs,
            dimension_numbers=(((0,), (0,)), ((), ())),
            preferred_element_type=jnp.float32)
        feather = jnp.maximum((acc + bcb).astype(jnp.bfloat16), 0)

        pooled2 = jnp.dot(feather, ones_ref[...],
                          preferred_element_type=jnp.float32)
        h1 = jnp.maximum(
            lax.dot_general(w1_ref[...], pooled2,
                            dimension_numbers=(((0,), (0,)), ((), ())),
                            preferred_element_type=jnp.float32) + b1c, 0.0)
        z = lax.dot_general(w2_ref[...], h1,
                            dimension_numbers=(((0,), (0,)), ((), ())),
                            preferred_element_type=jnp.float32)
        for j in range(G):
            gate = 1.0 + jax.nn.sigmoid(z[:, 128 * j:128 * j + 1] + b2c)
            out_ref[G * p + j] = (feather[:, HW * j:HW * (j + 1)]
                                  * gate.astype(jnp.bfloat16))


@jax.jit
def _ffm(sp, cx, wc, bc, bns, bnb, w1, b1, w2, b2):
    N, Cin, H, W = sp.shape
    Cout = w1.shape[0]
    HW = H * W
    Lpad = _round_up(HW + 2 * W + 2, 128)
    B = next(b for b in (32, 16, 8, 4, 2) if N % b == 0)
    G = 2

    bcb = bc.reshape(Cout) * bns.reshape(Cout) + bnb.reshape(Cout)
    bias_pack = jnp.stack([bcb, b1.reshape(Cout), b2.reshape(Cout)], axis=1)
    eyeg = jnp.eye(G, dtype=jnp.bfloat16)
    ones = jnp.kron(eyeg, jnp.full((HW, 128), 1.0 / HW, jnp.bfloat16))
    wc_r = wc.reshape(9 * Cin, Cout)

    x = (sp + cx).reshape(N, Cin, HW).astype(jnp.bfloat16)

    kernel_fn = functools.partial(_ffm_kernel, H=H, W=W, B=B, G=G)
    out = pl.pallas_call(
        kernel_fn,
        out_shape=jax.ShapeDtypeStruct((N, Cout, HW), jnp.bfloat16),
        grid=(N // B,),
        in_specs=[
            pl.BlockSpec((B, Cin, HW), lambda i: (i, 0, 0)),
            pl.BlockSpec((9 * Cin, Cout), lambda i: (0, 0)),
            pl.BlockSpec((1, Cout), lambda i: (0, 0)),
            pl.BlockSpec((Cout, 3), lambda i: (0, 0)),
            pl.BlockSpec((G * HW, 128 * G), lambda i: (0, 0)),
            pl.BlockSpec((Cout, Cout), lambda i: (0, 0)),
            pl.BlockSpec((Cout, Cout), lambda i: (0, 0)),
        ],
        out_specs=pl.BlockSpec((B, Cout, HW), lambda i: (i, 0, 0)),
        scratch_shapes=[pltpu.VMEM((2, G, Cin, Lpad), jnp.bfloat16)],
        compiler_params=pltpu.CompilerParams(
            dimension_semantics=("parallel",)),
        cost_estimate=pl.CostEstimate(
            flops=2 * N * 9 * Cout * Cin * HW + 2 * N * Cout * HW * 128
                  + 2 * N * 2 * Cout * Cout * 128,
            transcendentals=N * Cout,
            bytes_accessed=2 * (N * Cin * HW + N * Cout * HW)
                           + 4 * (2 * Cout * Cout + N * Cout)
                           + 4 * Cout * 9 * Cin),
    )(x, wc_r, bns, bias_pack, ones, w1, w2)

    return out.reshape(N, Cout, H, W).astype(jnp.float32)


def kernel(sp, cx, wc, bc, bns, bnb, w1, b1, w2, b2):
    return _ffm(sp, cx, wc, bc, bns, bnb, w1, b1, w2, b2)


# speedup vs baseline: 1.0621x; 1.0000x over previous
import functools

import jax
import jax.numpy as jnp
from jax import lax
from jax.experimental import pallas as pl
from jax.experimental.pallas import tpu as pltpu


def _round_up(x, m):
    return ((x + m - 1) // m) * m


def _ffm_kernel(x_ref, wc_ref, bns_ref, bias_ref, ones_ref, w1_ref, w2_ref,
                out_ref, xpad_ref, *, H, W, B, G):
    HW = H * W
    Cin = x_ref.shape[1]
    Cout = out_ref.shape[1]

    col = lax.broadcasted_iota(jnp.int32, (1, HW), 1) % W
    left_ok = col != 0
    right_ok = col != W - 1

    wk = (wc_ref[...].reshape(9 * Cin, Cout)
          * (0.5 * bns_ref[...])).astype(jnp.bfloat16)
    bcb = bias_ref[:, 0:1]
    b1c = bias_ref[:, 1:2]
    b2c = bias_ref[:, 2:3]

    base = W + 1
    Lpad = xpad_ref.shape[3]
    for slot in range(2):
        for j in range(G):
            xpad_ref[slot, j, :, 0:base] = jnp.zeros((Cin, base),
                                                     jnp.bfloat16)
            xpad_ref[slot, j, :, base + HW:Lpad] = jnp.zeros(
                (Cin, Lpad - base - HW), jnp.bfloat16)

    for p in range(B // G):
        slot = p % 2
        for j in range(G):
            xpad_ref[slot, j, :, base:base + HW] = x_ref[G * p + j]

        slices = []
        for kh in range(3):
            for kw in range(3):
                o = kh * W + kw
                pieces = []
                for j in range(G):
                    s = xpad_ref[slot, j, :, o:o + HW]
                    if kw == 0:
                        s = jnp.where(left_ok, s, 0)
                    elif kw == 2:
                        s = jnp.where(right_ok, s, 0)
                    pieces.append(s)
                slices.append(jnp.concatenate(pieces, axis=1))
        rhs = jnp.concatenate(slices, axis=0)

        acc = lax.dot_general(
            wk, rhs,
            dimension_numbers=(((0,), (0,)), ((), ())),
            preferred_element_type=jnp.float32)
        feather = jnp.maximum((acc + bcb).astype(jnp.bfloat16), 0)

        pooled2 = jnp.dot(feather, ones_ref[...],
                          preferred_element_type=jnp.float32)
        h1 = jnp.maximum(
            lax.dot_general(w1_ref[...], pooled2,
                            dimension_numbers=(((0,), (0,)), ((), ())),
                            preferred_element_type=jnp.float32) + b1c, 0.0)
        z = lax.dot_general(w2_ref[...], h1,
                            dimension_numbers=(((0,), (0,)), ((), ())),
                            preferred_element_type=jnp.float32)
        for j in range(G):
            gate = 1.0 + jax.nn.sigmoid(z[:, 128 * j:128 * j + 1] + b2c)
            out_ref[G * p + j] = (feather[:, HW * j:HW * (j + 1)]
                                  * gate.astype(jnp.bfloat16))


@jax.jit
def _ffm(sp, cx, wc, bc, bns, bnb, w1, b1, w2, b2):
    N, Cin, H, W = sp.shape
    Cout = w1.shape[0]
    HW = H * W
    Lpad = _round_up(HW + 2 * W + 2, 128)
    B = next(b for b in (16, 8, 4, 2) if N % b == 0)
    G = 2

    bcb = bc.reshape(Cout) * bns.reshape(Cout) + bnb.reshape(Cout)
    bias_pack = jnp.stack([bcb, b1.reshape(Cout), b2.reshape(Cout)], axis=1)
    eyeg = jnp.eye(G, dtype=jnp.bfloat16)
    ones = jnp.kron(eyeg, jnp.full((HW, 128), 1.0 / HW, jnp.bfloat16))
    wc_r = wc.reshape(9 * Cin, Cout)

    x = (sp + cx).reshape(N, Cin, HW).astype(jnp.bfloat16)

    kernel_fn = functools.partial(_ffm_kernel, H=H, W=W, B=B, G=G)
    out = pl.pallas_call(
        kernel_fn,
        out_shape=jax.ShapeDtypeStruct((N, Cout, HW), jnp.bfloat16),
        grid=(N // B,),
        in_specs=[
            pl.BlockSpec((B, Cin, HW), lambda i: (i, 0, 0)),
            pl.BlockSpec((9 * Cin, Cout), lambda i: (0, 0)),
            pl.BlockSpec((1, Cout), lambda i: (0, 0)),
            pl.BlockSpec((Cout, 3), lambda i: (0, 0)),
            pl.BlockSpec((G * HW, 128 * G), lambda i: (0, 0)),
            pl.BlockSpec((Cout, Cout), lambda i: (0, 0)),
            pl.BlockSpec((Cout, Cout), lambda i: (0, 0)),
        ],
        out_specs=pl.BlockSpec((B, Cout, HW), lambda i: (i, 0, 0)),
        scratch_shapes=[pltpu.VMEM((2, G, Cin, Lpad), jnp.bfloat16)],
        compiler_params=pltpu.CompilerParams(
            dimension_semantics=("parallel",)),
        cost_estimate=pl.CostEstimate(
            flops=2 * N * 9 * Cout * Cin * HW + 2 * N * Cout * HW * 128
                  + 2 * N * 2 * Cout * Cout * 128,
            transcendentals=N * Cout,
            bytes_accessed=2 * (N * Cin * HW + N * Cout * HW)
                           + 4 * (2 * Cout * Cout + N * Cout)
                           + 4 * Cout * 9 * Cin),
    )(x, wc_r, bns, bias_pack, ones, w1, w2)

    return out.reshape(N, Cout, H, W).astype(jnp.float32)


def kernel(sp, cx, wc, bc, bns, bnb, w1, b1, w2, b2):
    return _ffm(sp, cx, wc, bc, bns, bnb, w1, b1, w2, b2)
